# baseline (speedup 1.0000x reference)
import functools
import jax
import jax.numpy as jnp
from jax import lax
from jax.experimental import pallas as pl
from jax.experimental.pallas import tpu as pltpu
from jax.experimental.pallas import tpu_sc as plsc

_B = 16384
_D = 1024
_TC_ROWS = 10240
_SC_ROWS = _B - _TC_ROWS
_TC_BLK = 2048

_NC = 2
_NS = 16
_NW = _NC * _NS
_ROWS_PER_W = _SC_ROWS // _NW
_CH = 32
_NCH = _ROWS_PER_W // _CH
_LANES = 16
_VECS_PER_ROW = _D // _LANES
_RUNROLL = 8


def _tc_body(ts_ref, x_ref, w_ref, o_ref):
    t = ts_ref[0]
    row = w_ref[t, :]
    o_ref[...] = x_ref[...] + row[None, :]


def _tc_part(ts, x, W):
    return pl.pallas_call(
        _tc_body,
        grid_spec=pltpu.PrefetchScalarGridSpec(
            num_scalar_prefetch=1,
            grid=(_TC_ROWS // _TC_BLK,),
            in_specs=[
                pl.BlockSpec((_TC_BLK, _D), lambda i, ts: (i, 0)),
                pl.BlockSpec(W.shape, lambda i, ts: (0, 0)),
            ],
            out_specs=pl.BlockSpec((_TC_BLK, _D), lambda i, ts: (i, 0)),
        ),
        out_shape=jax.ShapeDtypeStruct((_TC_ROWS, _D), x.dtype),
    )(ts, x, W)


def _sc_body(ts_hbm, x_hbm, w_hbm, out_hbm, ts_v, w_v, buf_v, in_sem, out_sem):
    wid = lax.axis_index("s") * _NC + lax.axis_index("c")
    base = _TC_ROWS + wid * _ROWS_PER_W

    pltpu.sync_copy(ts_hbm, ts_v)
    t = ts_v[...][0]
    pltpu.sync_copy(w_hbm.at[t], w_v)

    def start_in(k, b):
        pltpu.make_async_copy(
            x_hbm.at[pl.ds(base + k * _CH, _CH)], buf_v.at[b], in_sem.at[b]
        ).start()

    def wait_in(k, b):
        pltpu.make_async_copy(
            x_hbm.at[pl.ds(base + k * _CH, _CH)], buf_v.at[b], in_sem.at[b]
        ).wait()

    def start_out(k, b):
        pltpu.make_async_copy(
            buf_v.at[b],
            out_hbm.at[pl.ds(wid * _ROWS_PER_W + k * _CH, _CH)],
            out_sem.at[b],
        ).start()

    def wait_out(k, b):
        pltpu.make_async_copy(
            buf_v.at[b],
            out_hbm.at[pl.ds(wid * _ROWS_PER_W + k * _CH, _CH)],
            out_sem.at[b],
        ).wait()

    def add_rows(b):
        def col(j, carry):
            sl = pl.ds(j * _LANES, _LANES)
            wv = w_v[sl]

            def rows(r0, carry2):
                for u in range(_RUNROLL):
                    r = r0 * _RUNROLL + u
                    buf_v[b, r, sl] = buf_v[b, r, sl] + wv
                return carry2

            lax.fori_loop(0, _CH // _RUNROLL, rows, 0)
            return carry

        lax.fori_loop(0, _VECS_PER_ROW, col, 0)

    start_in(0, 0)
    start_in(1, 1)

    def chunk_pair(i, carry):
        k0 = i * 2
        for b in range(2):
            k = k0 + b
            wait_in(k, b)
            add_rows(b)
            start_out(k, b)

            @pl.when(k + 2 < _NCH)
            def _():
                wait_out(k, b)
                start_in(k + 2, b)

        return carry

    lax.fori_loop(0, _NCH // 2, chunk_pair, 0)
    wait_out(_NCH - 2, 0)
    wait_out(_NCH - 1, 1)


def _sc_part(ts, x, W):
    mesh = plsc.VectorSubcoreMesh(core_axis_name="c", subcore_axis_name="s")
    f = functools.partial(
        pl.kernel,
        mesh=mesh,
        out_type=jax.ShapeDtypeStruct((_SC_ROWS, _D), jnp.float32),
        scratch_types=[
            pltpu.VMEM((16,), jnp.int32),
            pltpu.VMEM((_D,), jnp.float32),
            pltpu.VMEM((2, _CH, _D), jnp.float32),
            pltpu.SemaphoreType.DMA((2,)),
            pltpu.SemaphoreType.DMA((2,)),
        ],
    )(_sc_body)
    return f(ts, x, W)


def kernel(x, timestep, W):
    ts_tc = jnp.asarray(timestep, dtype=jnp.int32).reshape((1,))
    ts_sc = jnp.full((16,), timestep, dtype=jnp.int32)
    out_sc = _sc_part(ts_sc, x, W)
    out_tc = _tc_part(ts_tc, x, W)
    return jnp.concatenate([out_tc, out_sc], axis=0)


# speedup vs baseline: 1.4778x; 1.4778x over previous
import functools
import jax
import jax.numpy as jnp
from jax import lax
from jax.experimental import pallas as pl
from jax.experimental.pallas import tpu as pltpu
from jax.experimental.pallas import tpu_sc as plsc

_B = 16384
_D = 1024
_NC = 2
_NS = 16
_NW = _NC * _NS
_ROWS_PER_W = _B // _NW
_CH = 16
_NCH = _ROWS_PER_W // _CH
_NBUF = 4
_LANES = 16
_VECS_PER_ROW = _D // _LANES
_RUNROLL = 8


def _sc_body(ts_hbm, x_hbm, w_hbm, out_hbm, ts_v, w_v, buf_v, in_sem, out_sem):
    wid = lax.axis_index("s") * _NC + lax.axis_index("c")
    base = wid * _ROWS_PER_W

    def start_in(k, b):
        pltpu.make_async_copy(
            x_hbm.at[pl.ds(base + k * _CH, _CH)], buf_v.at[b], in_sem.at[b]
        ).start()

    def wait_in(k, b):
        pltpu.make_async_copy(
            x_hbm.at[pl.ds(base + k * _CH, _CH)], buf_v.at[b], in_sem.at[b]
        ).wait()

    def start_out(k, b):
        pltpu.make_async_copy(
            buf_v.at[b], out_hbm.at[pl.ds(base + k * _CH, _CH)], out_sem.at[b]
        ).start()

    def wait_out(k, b):
        pltpu.make_async_copy(
            buf_v.at[b], out_hbm.at[pl.ds(base + k * _CH, _CH)], out_sem.at[b]
        ).wait()

    for b in range(_NBUF):
        start_in(b, b)

    pltpu.sync_copy(ts_hbm, ts_v)
    t = ts_v[...][0]
    pltpu.sync_copy(w_hbm.at[pl.ds(t, 1)], w_v)

    def add_rows(b):
        def col(j, carry):
            sl = pl.ds(j * _LANES, _LANES)
            wv = w_v[0, sl]

            def rows(r0, carry2):
                for u in range(_RUNROLL):
                    r = r0 * _RUNROLL + u
                    buf_v[b, r, sl] = buf_v[b, r, sl] + wv
                return carry2

            lax.fori_loop(0, _CH // _RUNROLL, rows, 0)
            return carry

        lax.fori_loop(0, _VECS_PER_ROW, col, 0)

    def chunk_group(i, carry):
        k0 = i * _NBUF
        for b in range(_NBUF):
            k = k0 + b
            wait_in(k, b)
            add_rows(b)
            start_out(k, b)

            @pl.when(k + _NBUF < _NCH)
            def _():
                wait_out(k, b)
                start_in(k + _NBUF, b)

        return carry

    lax.fori_loop(0, _NCH // _NBUF, chunk_group, 0)
    for b in range(_NBUF):
        wait_out(_NCH - _NBUF + b, b)


def kernel(x, timestep, W):
    ts = jnp.full((16,), timestep, dtype=jnp.int32)
    mesh = plsc.VectorSubcoreMesh(core_axis_name="c", subcore_axis_name="s")
    f = functools.partial(
        pl.kernel,
        mesh=mesh,
        out_type=jax.ShapeDtypeStruct((_B, _D), jnp.float32),
        scratch_types=[
            pltpu.VMEM((16,), jnp.int32),
            pltpu.VMEM((1, _D), jnp.float32),
            pltpu.VMEM((_NBUF, _CH, _D), jnp.float32),
            pltpu.SemaphoreType.DMA((_NBUF,)),
            pltpu.SemaphoreType.DMA((_NBUF,)),
        ],
    )(_sc_body)
    return f(ts, x, W)


# speedup vs baseline: 2.3845x; 1.6136x over previous
import jax
import jax.numpy as jnp
from jax.experimental import pallas as pl
from jax.experimental.pallas import tpu as pltpu

_BLK = 2048


def _body(ts_ref, x_ref, w_ref, o_ref):
    o_ref[...] = x_ref[...] + w_ref[0, 0, :][None, :]


def kernel(x, timestep, W):
    B, D = x.shape
    ts = jnp.asarray(timestep, dtype=jnp.int32).reshape((1,))
    W3 = W.reshape(W.shape[0], 1, D)
    return pl.pallas_call(
        _body,
        grid_spec=pltpu.PrefetchScalarGridSpec(
            num_scalar_prefetch=1,
            grid=(B // _BLK,),
            in_specs=[
                pl.BlockSpec((_BLK, D), lambda i, ts: (i, 0)),
                pl.BlockSpec((1, 1, D), lambda i, ts: (ts[0], 0, 0)),
            ],
            out_specs=pl.BlockSpec((_BLK, D), lambda i, ts: (i, 0)),
        ),
        out_shape=jax.ShapeDtypeStruct((B, D), x.dtype),
    )(ts, x, W3)
